# Initial kernel scaffold; baseline (speedup 1.0000x reference)
import functools

import jax
import jax.numpy as jnp
from jax import lax
from jax.experimental import pallas as pl
from jax.experimental.pallas import tpu as pltpu
from jax.experimental.pallas import tpu_sc as plsc

N = 10000
D = 128
E = 320000
BATCH = 80
NB_TILE = E // (32 * BATCH)
ROWS_TILE = N // 16
CNT_PAD = 10240
CNT_TILE = CNT_PAD // 16

_MESH = plsc.VectorSubcoreMesh(core_axis_name="c", subcore_axis_name="s")


def _deg_body(dst_hbm, zero_hbm, cnt_out, dstv, onesv, cnt_sh):
    c = lax.axis_index("c")
    s = lax.axis_index("s")
    wid = s * 2 + c
    pltpu.sync_copy(zero_hbm.at[pl.ds(s * CNT_TILE, CNT_TILE)],
                    cnt_sh.at[pl.ds(s * CNT_TILE, CNT_TILE)])
    for i in range(BATCH // 16):
        onesv[pl.ds(i * 16, 16)] = jnp.ones((16,), jnp.float32)
    pltpu.sync_copy(dst_hbm.at[pl.ds(wid * NB_TILE, NB_TILE)], dstv)
    plsc.subcore_barrier()

    def body(j, carry):
        pltpu.sync_copy(onesv, cnt_sh.at[dstv.at[j]], add=True)
        return carry

    lax.fori_loop(0, NB_TILE, body, 0)
    plsc.subcore_barrier()
    pltpu.sync_copy(cnt_sh.at[pl.ds(s * CNT_TILE, CNT_TILE)],
                    cnt_out.at[c].at[pl.ds(s * CNT_TILE, CNT_TILE)])


_deg = pl.kernel(
    _deg_body,
    out_type=jax.ShapeDtypeStruct((2, CNT_PAD), jnp.float32),
    mesh=_MESH,
    scratch_types=[
        pltpu.VMEM((NB_TILE, BATCH), jnp.int32),
        pltpu.VMEM((BATCH,), jnp.float32),
        pltpu.VMEM_SHARED((CNT_PAD,), jnp.float32),
    ],
)


def _scat_body(z_hbm, src_hbm, dst_hbm, zero_hbm, out_hbm,
               srcv, dstv, rows, acc_sh, sem):
    c = lax.axis_index("c")
    s = lax.axis_index("s")
    wid = s * 2 + c
    pltpu.sync_copy(zero_hbm.at[pl.ds(s * ROWS_TILE, ROWS_TILE)],
                    acc_sh.at[pl.ds(s * ROWS_TILE, ROWS_TILE)])
    pltpu.sync_copy(src_hbm.at[pl.ds(wid * NB_TILE, NB_TILE)], srcv)
    pltpu.sync_copy(dst_hbm.at[pl.ds(wid * NB_TILE, NB_TILE)], dstv)
    plsc.subcore_barrier()

    def body(j, carry):
        pltpu.async_copy(z_hbm.at[srcv.at[j]], rows, sem).wait()
        pltpu.sync_copy(rows, acc_sh.at[dstv.at[j]], add=True)
        return carry

    lax.fori_loop(0, NB_TILE, body, 0)
    plsc.subcore_barrier()
    pltpu.sync_copy(acc_sh.at[pl.ds(s * ROWS_TILE, ROWS_TILE)],
                    out_hbm.at[c].at[pl.ds(s * ROWS_TILE, ROWS_TILE)])


_scat = pl.kernel(
    _scat_body,
    out_type=jax.ShapeDtypeStruct((2, N, D), jnp.float32),
    mesh=_MESH,
    scratch_types=[
        pltpu.VMEM((NB_TILE, BATCH), jnp.int32),
        pltpu.VMEM((NB_TILE, BATCH), jnp.int32),
        pltpu.VMEM((BATCH, D), jnp.float32),
        pltpu.VMEM_SHARED((N, D), jnp.float32),
        pltpu.SemaphoreType.DMA,
    ],
)

_R = 1000
_GRID = N // _R


def _embed_body(x_ref, we_ref, w1_ref, ca_ref, cb_ref, z1_ref, dinv_ref):
    dinv = lax.rsqrt(1.0 + ca_ref[...] + cb_ref[...])
    h0 = jnp.dot(x_ref[...], we_ref[...], preferred_element_type=jnp.float32)
    z1_ref[...] = jnp.dot(h0, w1_ref[...],
                          preferred_element_type=jnp.float32) * dinv
    dinv_ref[...] = dinv


def _embed(x, we, w1, ca, cb):
    return pl.pallas_call(
        _embed_body,
        grid=(_GRID,),
        in_specs=[
            pl.BlockSpec((_R, D), lambda i: (i, 0)),
            pl.BlockSpec((D, D), lambda i: (0, 0)),
            pl.BlockSpec((D, D), lambda i: (0, 0)),
            pl.BlockSpec((_R, 1), lambda i: (i, 0)),
            pl.BlockSpec((_R, 1), lambda i: (i, 0)),
        ],
        out_specs=[
            pl.BlockSpec((_R, D), lambda i: (i, 0)),
            pl.BlockSpec((_R, 1), lambda i: (i, 0)),
        ],
        out_shape=[
            jax.ShapeDtypeStruct((N, D), jnp.float32),
            jax.ShapeDtypeStruct((N, 1), jnp.float32),
        ],
    )(x, we, w1, ca, cb)


def _mid_body(aa_ref, ab_ref, z1_ref, dinv_ref, b1_ref, w2_ref, z2_ref):
    dinv = dinv_ref[...]
    h1 = jnp.maximum(
        (aa_ref[...] + ab_ref[...] + z1_ref[...]) * dinv + b1_ref[...], 0.0)
    z2_ref[...] = jnp.dot(h1, w2_ref[...],
                          preferred_element_type=jnp.float32) * dinv


def _mid(aa, ab, z1, dinv, b1, w2):
    return pl.pallas_call(
        _mid_body,
        grid=(_GRID,),
        in_specs=[
            pl.BlockSpec((_R, D), lambda i: (i, 0)),
            pl.BlockSpec((_R, D), lambda i: (i, 0)),
            pl.BlockSpec((_R, D), lambda i: (i, 0)),
            pl.BlockSpec((_R, 1), lambda i: (i, 0)),
            pl.BlockSpec((1, D), lambda i: (0, 0)),
            pl.BlockSpec((D, D), lambda i: (0, 0)),
        ],
        out_specs=pl.BlockSpec((_R, D), lambda i: (i, 0)),
        out_shape=jax.ShapeDtypeStruct((N, D), jnp.float32),
    )(aa, ab, z1, dinv, b1, w2)


def _head_body(aa_ref, ab_ref, z2_ref, dinv_ref, b2_ref, ct_ref, wo_ref,
               bo_ref, o_ref, acc_ref):
    i = pl.program_id(0)
    h2 = jnp.maximum(
        (aa_ref[...] + ab_ref[...] + z2_ref[...]) * dinv_ref[...]
        + b2_ref[...], 0.0)
    g = jnp.dot(h2, ct_ref[...], preferred_element_type=jnp.float32)
    rn = jnp.sum(h2 * h2, axis=1, keepdims=True)
    cn = jnp.sum(ct_ref[...] * ct_ref[...], axis=0, keepdims=True)
    dist = jnp.sqrt(jnp.clip(rn + cn - 2.0 * g, 1e-12, None))
    colsum = jnp.sum(dist, axis=0, keepdims=True)

    @pl.when(i == 0)
    def _():
        acc_ref[...] = jnp.zeros_like(acc_ref)

    acc_ref[...] += colsum

    @pl.when(i == pl.num_programs(0) - 1)
    def _():
        o_ref[...] = jnp.dot(acc_ref[...] * (1.0 / N), wo_ref[...],
                             preferred_element_type=jnp.float32) + bo_ref[...]


def _head(aa, ab, z2, dinv, b2, ct, wo, bo):
    return pl.pallas_call(
        _head_body,
        grid=(_GRID,),
        in_specs=[
            pl.BlockSpec((_R, D), lambda i: (i, 0)),
            pl.BlockSpec((_R, D), lambda i: (i, 0)),
            pl.BlockSpec((_R, D), lambda i: (i, 0)),
            pl.BlockSpec((_R, 1), lambda i: (i, 0)),
            pl.BlockSpec((1, D), lambda i: (0, 0)),
            pl.BlockSpec((D, D), lambda i: (0, 0)),
            pl.BlockSpec((D, D), lambda i: (0, 0)),
            pl.BlockSpec((1, D), lambda i: (0, 0)),
        ],
        out_specs=pl.BlockSpec((1, D), lambda i: (0, 0)),
        out_shape=jax.ShapeDtypeStruct((1, D), jnp.float32),
        scratch_shapes=[pltpu.VMEM((1, D), jnp.float32)],
    )(aa, ab, z2, dinv, b2, ct, wo, bo)


def kernel(x, edge_index, W_embed, W1, b1, W2, b2, centroids, W_out, b_out):
    src2 = edge_index[0].astype(jnp.int32).reshape(E // BATCH, BATCH)
    dst2 = edge_index[1].astype(jnp.int32).reshape(E // BATCH, BATCH)
    zf = jnp.zeros((N, D), jnp.float32)
    zc = jnp.zeros((CNT_PAD,), jnp.float32)

    cnt = _deg(dst2, zc)
    ca = cnt[0, :N].reshape(N, 1)
    cb = cnt[1, :N].reshape(N, 1)

    z1, dinv = _embed(x, W_embed, W1, ca, cb)
    acc1 = _scat(z1, src2, dst2, zf)
    z2 = _mid(acc1[0], acc1[1], z1, dinv, b1.reshape(1, D), W2)
    acc2 = _scat(z2, src2, dst2, zf)

    cpad = jnp.zeros((D, D), jnp.float32).at[:centroids.shape[0]].set(centroids)
    wo = jnp.zeros((D, D), jnp.float32).at[:W_out.shape[0], :W_out.shape[1]].set(W_out)
    bo = jnp.zeros((1, D), jnp.float32).at[0, :b_out.shape[0]].set(b_out)
    o = _head(acc2[0], acc2[1], z2, dinv, b2.reshape(1, D), cpad.T, wo, bo)
    return o[0, :b_out.shape[0]]


# speedup vs baseline: 19.0045x; 19.0045x over previous
import functools

import jax
import jax.numpy as jnp
from jax import lax
from jax.experimental import pallas as pl
from jax.experimental.pallas import tpu as pltpu
from jax.experimental.pallas import tpu_sc as plsc

N = 10000
D = 128
E = 320000
BATCH = 80
NB_TILE = E // (32 * BATCH)
N_PAD = 10240
ROWS_TILE = N_PAD // 16
CNT_PAD = 10240
CNT_TILE = CNT_PAD // 16

_MESH = plsc.VectorSubcoreMesh(core_axis_name="c", subcore_axis_name="s")


def _deg_body(dst_hbm, zero_hbm, cnt_out, dstv, onesv, cnt_sh):
    c = lax.axis_index("c")
    s = lax.axis_index("s")
    wid = s * 2 + c
    pltpu.sync_copy(zero_hbm.at[pl.ds(s * CNT_TILE, CNT_TILE)],
                    cnt_sh.at[pl.ds(s * CNT_TILE, CNT_TILE)])
    for i in range(BATCH // 16):
        onesv[pl.ds(i * 16, 16)] = jnp.ones((16,), jnp.float32)
    pltpu.sync_copy(dst_hbm.at[wid], dstv)
    plsc.subcore_barrier()

    def body(j, carry):
        pltpu.sync_copy(onesv, cnt_sh.at[dstv.at[j]], add=True)
        return carry

    lax.fori_loop(0, NB_TILE, body, 0)
    plsc.subcore_barrier()
    pltpu.sync_copy(cnt_sh.at[pl.ds(s * CNT_TILE, CNT_TILE)],
                    cnt_out.at[c].at[pl.ds(s * CNT_TILE, CNT_TILE)])


_deg = pl.kernel(
    _deg_body,
    out_type=jax.ShapeDtypeStruct((2, CNT_PAD), jnp.float32),
    mesh=_MESH,
    scratch_types=[
        pltpu.VMEM((NB_TILE, BATCH), jnp.int32),
        pltpu.VMEM((BATCH,), jnp.float32),
        pltpu.VMEM_SHARED((CNT_PAD,), jnp.float32),
    ],
)


def _scat_body(z_hbm, src_hbm, dst_hbm, zero_hbm, out_hbm,
               srcv, dstv, rows, acc_sh, sem):
    c = lax.axis_index("c")
    s = lax.axis_index("s")
    wid = s * 2 + c
    pltpu.sync_copy(zero_hbm.at[pl.ds(s * ROWS_TILE, ROWS_TILE)],
                    acc_sh.at[pl.ds(s * ROWS_TILE, ROWS_TILE)])
    pltpu.sync_copy(src_hbm.at[wid], srcv)
    pltpu.sync_copy(dst_hbm.at[wid], dstv)
    plsc.subcore_barrier()

    def body(j, carry):
        pltpu.async_copy(z_hbm.at[srcv.at[j]], rows, sem).wait()
        pltpu.sync_copy(rows, acc_sh.at[dstv.at[j]], add=True)
        return carry

    lax.fori_loop(0, NB_TILE, body, 0)
    plsc.subcore_barrier()
    pltpu.sync_copy(acc_sh.at[pl.ds(s * ROWS_TILE, ROWS_TILE)],
                    out_hbm.at[c].at[pl.ds(s * ROWS_TILE, ROWS_TILE)])


_scat = pl.kernel(
    _scat_body,
    out_type=jax.ShapeDtypeStruct((2, N_PAD, D), jnp.float32),
    mesh=_MESH,
    scratch_types=[
        pltpu.VMEM((NB_TILE, BATCH), jnp.int32),
        pltpu.VMEM((NB_TILE, BATCH), jnp.int32),
        pltpu.VMEM((BATCH, D), jnp.float32),
        pltpu.VMEM_SHARED((N_PAD, D), jnp.float32),
        pltpu.SemaphoreType.DMA,
    ],
)

_R = 1000
_GRID = N // _R


def _embed_body(x_ref, we_ref, w1_ref, ca_ref, cb_ref, z1_ref, dinv_ref):
    dinv = lax.rsqrt(1.0 + ca_ref[...] + cb_ref[...])
    h0 = jnp.dot(x_ref[...], we_ref[...], preferred_element_type=jnp.float32)
    z1_ref[...] = jnp.dot(h0, w1_ref[...],
                          preferred_element_type=jnp.float32) * dinv
    dinv_ref[...] = dinv


def _embed(x, we, w1, ca, cb):
    return pl.pallas_call(
        _embed_body,
        grid=(_GRID,),
        in_specs=[
            pl.BlockSpec((_R, D), lambda i: (i, 0)),
            pl.BlockSpec((D, D), lambda i: (0, 0)),
            pl.BlockSpec((D, D), lambda i: (0, 0)),
            pl.BlockSpec((_R, 1), lambda i: (i, 0)),
            pl.BlockSpec((_R, 1), lambda i: (i, 0)),
        ],
        out_specs=[
            pl.BlockSpec((_R, D), lambda i: (i, 0)),
            pl.BlockSpec((_R, 1), lambda i: (i, 0)),
        ],
        out_shape=[
            jax.ShapeDtypeStruct((N, D), jnp.float32),
            jax.ShapeDtypeStruct((N, 1), jnp.float32),
        ],
    )(x, we, w1, ca, cb)


def _mid_body(aa_ref, ab_ref, z1_ref, dinv_ref, b1_ref, w2_ref, z2_ref):
    dinv = dinv_ref[...]
    h1 = jnp.maximum(
        (aa_ref[...] + ab_ref[...] + z1_ref[...]) * dinv + b1_ref[...], 0.0)
    z2_ref[...] = jnp.dot(h1, w2_ref[...],
                          preferred_element_type=jnp.float32) * dinv


def _mid(aa, ab, z1, dinv, b1, w2):
    return pl.pallas_call(
        _mid_body,
        grid=(_GRID,),
        in_specs=[
            pl.BlockSpec((_R, D), lambda i: (i, 0)),
            pl.BlockSpec((_R, D), lambda i: (i, 0)),
            pl.BlockSpec((_R, D), lambda i: (i, 0)),
            pl.BlockSpec((_R, 1), lambda i: (i, 0)),
            pl.BlockSpec((1, D), lambda i: (0, 0)),
            pl.BlockSpec((D, D), lambda i: (0, 0)),
        ],
        out_specs=pl.BlockSpec((_R, D), lambda i: (i, 0)),
        out_shape=jax.ShapeDtypeStruct((N, D), jnp.float32),
    )(aa, ab, z1, dinv, b1, w2)


def _head_body(aa_ref, ab_ref, z2_ref, dinv_ref, b2_ref, ct_ref, wo_ref,
               bo_ref, o_ref, acc_ref):
    i = pl.program_id(0)
    h2 = jnp.maximum(
        (aa_ref[...] + ab_ref[...] + z2_ref[...]) * dinv_ref[...]
        + b2_ref[...], 0.0)
    g = jnp.dot(h2, ct_ref[...], preferred_element_type=jnp.float32)
    rn = jnp.sum(h2 * h2, axis=1, keepdims=True)
    cn = jnp.sum(ct_ref[...] * ct_ref[...], axis=0, keepdims=True)
    dist = jnp.sqrt(jnp.clip(rn + cn - 2.0 * g, 1e-12, None))
    colsum = jnp.sum(dist, axis=0, keepdims=True)

    @pl.when(i == 0)
    def _():
        acc_ref[...] = jnp.zeros_like(acc_ref)

    acc_ref[...] += colsum

    @pl.when(i == pl.num_programs(0) - 1)
    def _():
        o_ref[...] = jnp.dot(acc_ref[...] * (1.0 / N), wo_ref[...],
                             preferred_element_type=jnp.float32) + bo_ref[...]


def _head(aa, ab, z2, dinv, b2, ct, wo, bo):
    return pl.pallas_call(
        _head_body,
        grid=(_GRID,),
        in_specs=[
            pl.BlockSpec((_R, D), lambda i: (i, 0)),
            pl.BlockSpec((_R, D), lambda i: (i, 0)),
            pl.BlockSpec((_R, D), lambda i: (i, 0)),
            pl.BlockSpec((_R, 1), lambda i: (i, 0)),
            pl.BlockSpec((1, D), lambda i: (0, 0)),
            pl.BlockSpec((D, D), lambda i: (0, 0)),
            pl.BlockSpec((D, D), lambda i: (0, 0)),
            pl.BlockSpec((1, D), lambda i: (0, 0)),
        ],
        out_specs=pl.BlockSpec((1, D), lambda i: (0, 0)),
        out_shape=jax.ShapeDtypeStruct((1, D), jnp.float32),
        scratch_shapes=[pltpu.VMEM((1, D), jnp.float32)],
    )(aa, ab, z2, dinv, b2, ct, wo, bo)


def kernel(x, edge_index, W_embed, W1, b1, W2, b2, centroids, W_out, b_out):
    src2 = edge_index[0].astype(jnp.int32).reshape(32, NB_TILE, BATCH)
    dst2 = edge_index[1].astype(jnp.int32).reshape(32, NB_TILE, BATCH)
    zf = jnp.zeros((N_PAD, D), jnp.float32)
    zc = jnp.zeros((CNT_PAD,), jnp.float32)

    cnt = _deg(dst2, zc)
    ca = cnt[0, :N].reshape(N, 1)
    cb = cnt[1, :N].reshape(N, 1)

    z1, dinv = _embed(x, W_embed, W1, ca, cb)
    acc1 = _scat(z1, src2, dst2, zf)
    z2 = _mid(acc1[0, :N], acc1[1, :N], z1, dinv, b1.reshape(1, D), W2)
    acc2 = _scat(z2, src2, dst2, zf)

    cpad = jnp.zeros((D, D), jnp.float32).at[:centroids.shape[0]].set(centroids)
    wo = jnp.zeros((D, D), jnp.float32).at[:W_out.shape[0], :W_out.shape[1]].set(W_out)
    bo = jnp.zeros((1, D), jnp.float32).at[0, :b_out.shape[0]].set(b_out)
    o = _head(acc2[0, :N], acc2[1, :N], z2, dinv, b2.reshape(1, D), cpad.T, wo, bo)
    return o[0, :b_out.shape[0]]


# speedup vs baseline: 27.2241x; 1.4325x over previous
import functools

import jax
import jax.numpy as jnp
from jax import lax
from jax.experimental import pallas as pl
from jax.experimental.pallas import tpu as pltpu
from jax.experimental.pallas import tpu_sc as plsc

N = 10000
D = 128
E = 320000
BATCH = 80
NB_TILE = E // (32 * BATCH)
N_PAD = 10112
ROWS_TILE = N_PAD // 16
CHUNK = 25
CNT_PAD = 10240
CNT_TILE = CNT_PAD // 16

_MESH = plsc.VectorSubcoreMesh(core_axis_name="c", subcore_axis_name="s")


def _deg_body(dst_hbm, zero_hbm, cnt_out, dstv, onesv, cnt_sh):
    c = lax.axis_index("c")
    s = lax.axis_index("s")
    wid = s * 2 + c
    pltpu.sync_copy(zero_hbm.at[pl.ds(s * CNT_TILE, CNT_TILE)],
                    cnt_sh.at[pl.ds(s * CNT_TILE, CNT_TILE)])
    for i in range(BATCH // 16):
        onesv[pl.ds(i * 16, 16)] = jnp.ones((16,), jnp.float32)
    pltpu.sync_copy(dst_hbm.at[wid], dstv)
    plsc.subcore_barrier()

    def body(j, carry):
        pltpu.sync_copy(onesv, cnt_sh.at[dstv.at[j // CHUNK].at[j % CHUNK]],
                        add=True)
        return carry

    lax.fori_loop(0, NB_TILE, body, 0)
    plsc.subcore_barrier()
    pltpu.sync_copy(cnt_sh.at[pl.ds(s * CNT_TILE, CNT_TILE)],
                    cnt_out.at[c].at[pl.ds(s * CNT_TILE, CNT_TILE)])


_deg = pl.kernel(
    _deg_body,
    out_type=jax.ShapeDtypeStruct((2, CNT_PAD), jnp.float32),
    mesh=_MESH,
    scratch_types=[
        pltpu.VMEM((NB_TILE // CHUNK, CHUNK, BATCH), jnp.int32),
        pltpu.VMEM((BATCH,), jnp.float32),
        pltpu.VMEM_SHARED((CNT_PAD,), jnp.float32),
    ],
)


def _scat_body(z_hbm, src_hbm, dst_hbm, zero_hbm, out_hbm,
               srcv, dstv, rows, acc_sh, sem0, sem1):
    c = lax.axis_index("c")
    s = lax.axis_index("s")
    wid = s * 2 + c
    pltpu.sync_copy(zero_hbm.at[pl.ds(s * ROWS_TILE, ROWS_TILE)],
                    acc_sh.at[pl.ds(s * ROWS_TILE, ROWS_TILE)])
    plsc.subcore_barrier()

    def chunk(c5, carry):
        pltpu.sync_copy(src_hbm.at[wid].at[c5], srcv)
        pltpu.sync_copy(dst_hbm.at[wid].at[c5], dstv)
        pltpu.async_copy(z_hbm.at[srcv.at[0]], rows.at[0], sem0)
        pltpu.async_copy(z_hbm.at[srcv.at[1]], rows.at[1], sem1)

        def pair(p, carry2):
            for b, sem in ((0, sem0), (1, sem1)):
                j = 2 * p + b
                pltpu.make_async_copy(z_hbm.at[srcv.at[j]], rows.at[b],
                                      sem).wait()
                pltpu.sync_copy(rows.at[b], acc_sh.at[dstv.at[j]], add=True)
                nxt = j + 2

                @pl.when(nxt < CHUNK)
                def _():
                    pltpu.async_copy(z_hbm.at[srcv.at[nxt]], rows.at[b], sem)
            return carry2

        lax.fori_loop(0, CHUNK // 2, pair, 0)
        last = CHUNK - 1
        pltpu.make_async_copy(z_hbm.at[srcv.at[last]], rows.at[0], sem0).wait()
        pltpu.sync_copy(rows.at[0], acc_sh.at[dstv.at[last]], add=True)
        return carry

    lax.fori_loop(0, NB_TILE // CHUNK, chunk, 0)
    plsc.subcore_barrier()
    pltpu.sync_copy(acc_sh.at[pl.ds(s * ROWS_TILE, ROWS_TILE)],
                    out_hbm.at[c].at[pl.ds(s * ROWS_TILE, ROWS_TILE)])


_scat = pl.kernel(
    _scat_body,
    out_type=jax.ShapeDtypeStruct((2, N_PAD, D), jnp.float32),
    mesh=_MESH,
    scratch_types=[
        pltpu.VMEM((CHUNK, BATCH), jnp.int32),
        pltpu.VMEM((CHUNK, BATCH), jnp.int32),
        pltpu.VMEM((2, BATCH, D), jnp.float32),
        pltpu.VMEM_SHARED((N_PAD, D), jnp.float32),
        pltpu.SemaphoreType.DMA,
        pltpu.SemaphoreType.DMA,
    ],
)

_R = 1000
_GRID = N // _R


def _embed_body(x_ref, we_ref, w1_ref, ca_ref, cb_ref, z1_ref, dinv_ref):
    dinv = lax.rsqrt(1.0 + ca_ref[...] + cb_ref[...])
    h0 = jnp.dot(x_ref[...], we_ref[...], preferred_element_type=jnp.float32)
    z1_ref[...] = jnp.dot(h0, w1_ref[...],
                          preferred_element_type=jnp.float32) * dinv
    dinv_ref[...] = dinv


def _embed(x, we, w1, ca, cb):
    return pl.pallas_call(
        _embed_body,
        grid=(_GRID,),
        in_specs=[
            pl.BlockSpec((_R, D), lambda i: (i, 0)),
            pl.BlockSpec((D, D), lambda i: (0, 0)),
            pl.BlockSpec((D, D), lambda i: (0, 0)),
            pl.BlockSpec((_R, 1), lambda i: (i, 0)),
            pl.BlockSpec((_R, 1), lambda i: (i, 0)),
        ],
        out_specs=[
            pl.BlockSpec((_R, D), lambda i: (i, 0)),
            pl.BlockSpec((_R, 1), lambda i: (i, 0)),
        ],
        out_shape=[
            jax.ShapeDtypeStruct((N, D), jnp.float32),
            jax.ShapeDtypeStruct((N, 1), jnp.float32),
        ],
    )(x, we, w1, ca, cb)


def _mid_body(aa_ref, ab_ref, z1_ref, dinv_ref, b1_ref, w2_ref, z2_ref):
    dinv = dinv_ref[...]
    h1 = jnp.maximum(
        (aa_ref[...] + ab_ref[...] + z1_ref[...]) * dinv + b1_ref[...], 0.0)
    z2_ref[...] = jnp.dot(h1, w2_ref[...],
                          preferred_element_type=jnp.float32) * dinv


def _mid(aa, ab, z1, dinv, b1, w2):
    return pl.pallas_call(
        _mid_body,
        grid=(_GRID,),
        in_specs=[
            pl.BlockSpec((_R, D), lambda i: (i, 0)),
            pl.BlockSpec((_R, D), lambda i: (i, 0)),
            pl.BlockSpec((_R, D), lambda i: (i, 0)),
            pl.BlockSpec((_R, 1), lambda i: (i, 0)),
            pl.BlockSpec((1, D), lambda i: (0, 0)),
            pl.BlockSpec((D, D), lambda i: (0, 0)),
        ],
        out_specs=pl.BlockSpec((_R, D), lambda i: (i, 0)),
        out_shape=jax.ShapeDtypeStruct((N, D), jnp.float32),
    )(aa, ab, z1, dinv, b1, w2)


def _head_body(aa_ref, ab_ref, z2_ref, dinv_ref, b2_ref, ct_ref, wo_ref,
               bo_ref, o_ref, acc_ref):
    i = pl.program_id(0)
    h2 = jnp.maximum(
        (aa_ref[...] + ab_ref[...] + z2_ref[...]) * dinv_ref[...]
        + b2_ref[...], 0.0)
    g = jnp.dot(h2, ct_ref[...], preferred_element_type=jnp.float32)
    rn = jnp.sum(h2 * h2, axis=1, keepdims=True)
    cn = jnp.sum(ct_ref[...] * ct_ref[...], axis=0, keepdims=True)
    dist = jnp.sqrt(jnp.clip(rn + cn - 2.0 * g, 1e-12, None))
    colsum = jnp.sum(dist, axis=0, keepdims=True)

    @pl.when(i == 0)
    def _():
        acc_ref[...] = jnp.zeros_like(acc_ref)

    acc_ref[...] += colsum

    @pl.when(i == pl.num_programs(0) - 1)
    def _():
        o_ref[...] = jnp.dot(acc_ref[...] * (1.0 / N), wo_ref[...],
                             preferred_element_type=jnp.float32) + bo_ref[...]


def _head(aa, ab, z2, dinv, b2, ct, wo, bo):
    return pl.pallas_call(
        _head_body,
        grid=(_GRID,),
        in_specs=[
            pl.BlockSpec((_R, D), lambda i: (i, 0)),
            pl.BlockSpec((_R, D), lambda i: (i, 0)),
            pl.BlockSpec((_R, D), lambda i: (i, 0)),
            pl.BlockSpec((_R, 1), lambda i: (i, 0)),
            pl.BlockSpec((1, D), lambda i: (0, 0)),
            pl.BlockSpec((D, D), lambda i: (0, 0)),
            pl.BlockSpec((D, D), lambda i: (0, 0)),
            pl.BlockSpec((1, D), lambda i: (0, 0)),
        ],
        out_specs=pl.BlockSpec((1, D), lambda i: (0, 0)),
        out_shape=jax.ShapeDtypeStruct((1, D), jnp.float32),
        scratch_shapes=[pltpu.VMEM((1, D), jnp.float32)],
    )(aa, ab, z2, dinv, b2, ct, wo, bo)


def kernel(x, edge_index, W_embed, W1, b1, W2, b2, centroids, W_out, b_out):
    src2 = edge_index[0].astype(jnp.int32).reshape(32, NB_TILE // CHUNK,
                                                    CHUNK, BATCH)
    dst2 = edge_index[1].astype(jnp.int32).reshape(32, NB_TILE // CHUNK,
                                                    CHUNK, BATCH)
    zf = jnp.zeros((N_PAD, D), jnp.float32)
    zc = jnp.zeros((CNT_PAD,), jnp.float32)

    cnt = _deg(dst2, zc)
    ca = cnt[0, :N].reshape(N, 1)
    cb = cnt[1, :N].reshape(N, 1)

    z1, dinv = _embed(x, W_embed, W1, ca, cb)
    acc1 = _scat(z1, src2, dst2, zf)
    z2 = _mid(acc1[0, :N], acc1[1, :N], z1, dinv, b1.reshape(1, D), W2)
    acc2 = _scat(z2, src2, dst2, zf)

    cpad = jnp.zeros((D, D), jnp.float32).at[:centroids.shape[0]].set(centroids)
    wo = jnp.zeros((D, D), jnp.float32).at[:W_out.shape[0], :W_out.shape[1]].set(W_out)
    bo = jnp.zeros((1, D), jnp.float32).at[0, :b_out.shape[0]].set(b_out)
    o = _head(acc2[0, :N], acc2[1, :N], z2, dinv, b2.reshape(1, D), cpad.T, wo, bo)
    return o[0, :b_out.shape[0]]


# speedup vs baseline: 29.3248x; 1.0772x over previous
import functools

import jax
import jax.numpy as jnp
from jax import lax
from jax.experimental import pallas as pl
from jax.experimental.pallas import tpu as pltpu
from jax.experimental.pallas import tpu_sc as plsc

N = 10000
D = 128
E = 320000
BATCH = 80
NB_TILE = E // (32 * BATCH)
N_PAD = 10112
ROWS_TILE = N_PAD // 16
CHUNK = 25
CNT_PAD = 10240
CNT_TILE = CNT_PAD // 16

_MESH = plsc.VectorSubcoreMesh(core_axis_name="c", subcore_axis_name="s")


def _deg_body(ei_hbm, zero_hbm, cnt_out, dstv, onesv, cnt_sh):
    c = lax.axis_index("c")
    s = lax.axis_index("s")
    wid = s * 2 + c
    pltpu.sync_copy(zero_hbm.at[pl.ds(s * CNT_TILE, CNT_TILE)],
                    cnt_sh.at[pl.ds(s * CNT_TILE, CNT_TILE)])
    for i in range(BATCH // 16):
        onesv[pl.ds(i * 16, 16)] = jnp.ones((16,), jnp.float32)
    pltpu.sync_copy(ei_hbm.at[1].at[wid], dstv)
    plsc.subcore_barrier()

    def body(j, carry):
        pltpu.sync_copy(onesv, cnt_sh.at[dstv.at[j // CHUNK].at[j % CHUNK]],
                        add=True)
        return carry

    lax.fori_loop(0, NB_TILE, body, 0)
    plsc.subcore_barrier()
    pltpu.sync_copy(cnt_sh.at[pl.ds(s * CNT_TILE, CNT_TILE)],
                    cnt_out.at[c].at[pl.ds(s * CNT_TILE, CNT_TILE)])


_deg = pl.kernel(
    _deg_body,
    out_type=jax.ShapeDtypeStruct((2, CNT_PAD), jnp.float32),
    mesh=_MESH,
    scratch_types=[
        pltpu.VMEM((NB_TILE // CHUNK, CHUNK, BATCH), jnp.int32),
        pltpu.VMEM((BATCH,), jnp.float32),
        pltpu.VMEM_SHARED((CNT_PAD,), jnp.float32),
    ],
)


def _scat_body(z_hbm, ei_hbm, zero_hbm, out_hbm,
               srcv, dstv, rows, acc_sh, sem0, sem1):
    c = lax.axis_index("c")
    s = lax.axis_index("s")
    wid = s * 2 + c
    pltpu.sync_copy(zero_hbm.at[pl.ds(s * ROWS_TILE, ROWS_TILE)],
                    acc_sh.at[pl.ds(s * ROWS_TILE, ROWS_TILE)])
    plsc.subcore_barrier()

    def chunk(c5, carry):
        pltpu.sync_copy(ei_hbm.at[0].at[wid].at[c5], srcv)
        pltpu.sync_copy(ei_hbm.at[1].at[wid].at[c5], dstv)
        pltpu.async_copy(z_hbm.at[srcv.at[0]], rows.at[0], sem0)
        pltpu.async_copy(z_hbm.at[srcv.at[1]], rows.at[1], sem1)

        def pair(p, carry2):
            for b, sem in ((0, sem0), (1, sem1)):
                j = 2 * p + b
                pltpu.make_async_copy(z_hbm.at[srcv.at[j]], rows.at[b],
                                      sem).wait()
                pltpu.sync_copy(rows.at[b], acc_sh.at[dstv.at[j]], add=True)
                nxt = j + 2

                @pl.when(nxt < CHUNK)
                def _():
                    pltpu.async_copy(z_hbm.at[srcv.at[nxt]], rows.at[b], sem)
            return carry2

        lax.fori_loop(0, CHUNK // 2, pair, 0)
        last = CHUNK - 1
        pltpu.make_async_copy(z_hbm.at[srcv.at[last]], rows.at[0], sem0).wait()
        pltpu.sync_copy(rows.at[0], acc_sh.at[dstv.at[last]], add=True)
        return carry

    lax.fori_loop(0, NB_TILE // CHUNK, chunk, 0)
    plsc.subcore_barrier()
    pltpu.sync_copy(acc_sh.at[pl.ds(s * ROWS_TILE, ROWS_TILE)],
                    out_hbm.at[c].at[pl.ds(s * ROWS_TILE, ROWS_TILE)])


_scat = pl.kernel(
    _scat_body,
    out_type=jax.ShapeDtypeStruct((2, N_PAD, D), jnp.float32),
    mesh=_MESH,
    scratch_types=[
        pltpu.VMEM((CHUNK, BATCH), jnp.int32),
        pltpu.VMEM((CHUNK, BATCH), jnp.int32),
        pltpu.VMEM((2, BATCH, D), jnp.float32),
        pltpu.VMEM_SHARED((N_PAD, D), jnp.float32),
        pltpu.SemaphoreType.DMA,
        pltpu.SemaphoreType.DMA,
    ],
)

_R = 1000
_GRID = N // _R


def _pre_body(x_ref, we_ref, w1_ref, y_ref):
    h0 = jnp.dot(x_ref[...], we_ref[...], preferred_element_type=jnp.float32)
    y_ref[...] = jnp.dot(h0, w1_ref[...], preferred_element_type=jnp.float32)


def _pre(x, we, w1):
    return pl.pallas_call(
        _pre_body,
        grid=(_GRID,),
        in_specs=[
            pl.BlockSpec((_R, D), lambda i: (i, 0)),
            pl.BlockSpec((D, D), lambda i: (0, 0)),
            pl.BlockSpec((D, D), lambda i: (0, 0)),
        ],
        out_specs=pl.BlockSpec((_R, D), lambda i: (i, 0)),
        out_shape=jax.ShapeDtypeStruct((N, D), jnp.float32),
    )(x, we, w1)


def _scale_body(y_ref, cnt_ref, z1_ref, dinv_ref):
    dinv = lax.rsqrt(1.0 + cnt_ref[0] + cnt_ref[1])
    z1_ref[...] = y_ref[...] * dinv
    dinv_ref[...] = dinv


def _scale(y, cnt3):
    return pl.pallas_call(
        _scale_body,
        grid=(_GRID,),
        in_specs=[
            pl.BlockSpec((_R, D), lambda i: (i, 0)),
            pl.BlockSpec((2, _R, 1), lambda i: (0, i, 0)),
        ],
        out_specs=[
            pl.BlockSpec((_R, D), lambda i: (i, 0)),
            pl.BlockSpec((_R, 1), lambda i: (i, 0)),
        ],
        out_shape=[
            jax.ShapeDtypeStruct((N, D), jnp.float32),
            jax.ShapeDtypeStruct((N, 1), jnp.float32),
        ],
    )(y, cnt3)


def _mid_body(aa_ref, ab_ref, z1_ref, dinv_ref, b1_ref, w2_ref, z2_ref):
    dinv = dinv_ref[...]
    h1 = jnp.maximum(
        (aa_ref[0] + ab_ref[0] + z1_ref[...]) * dinv + b1_ref[...], 0.0)
    z2_ref[...] = jnp.dot(h1, w2_ref[...],
                          preferred_element_type=jnp.float32) * dinv


def _mid(acc, z1, dinv, b1, w2):
    return pl.pallas_call(
        _mid_body,
        grid=(_GRID,),
        in_specs=[
            pl.BlockSpec((1, _R, D), lambda i: (0, i, 0)),
            pl.BlockSpec((1, _R, D), lambda i: (1, i, 0)),
            pl.BlockSpec((_R, D), lambda i: (i, 0)),
            pl.BlockSpec((_R, 1), lambda i: (i, 0)),
            pl.BlockSpec((1, D), lambda i: (0, 0)),
            pl.BlockSpec((D, D), lambda i: (0, 0)),
        ],
        out_specs=pl.BlockSpec((_R, D), lambda i: (i, 0)),
        out_shape=jax.ShapeDtypeStruct((N, D), jnp.float32),
    )(acc, acc, z1, dinv, b1, w2)


def _head_body(aa_ref, ab_ref, z2_ref, dinv_ref, b2_ref, ct_ref, wo_ref,
               bo_ref, o_ref, acc_ref):
    i = pl.program_id(0)
    h2 = jnp.maximum(
        (aa_ref[0] + ab_ref[0] + z2_ref[...]) * dinv_ref[...]
        + b2_ref[...], 0.0)
    g = jnp.dot(h2, ct_ref[...], preferred_element_type=jnp.float32)
    rn = jnp.sum(h2 * h2, axis=1, keepdims=True)
    cn = jnp.sum(ct_ref[...] * ct_ref[...], axis=0, keepdims=True)
    dist = jnp.sqrt(jnp.clip(rn + cn - 2.0 * g, 1e-12, None))
    colsum = jnp.sum(dist, axis=0, keepdims=True)

    @pl.when(i == 0)
    def _():
        acc_ref[...] = jnp.zeros_like(acc_ref)

    acc_ref[...] += colsum

    @pl.when(i == pl.num_programs(0) - 1)
    def _():
        o_ref[...] = jnp.dot(acc_ref[...] * (1.0 / N), wo_ref[...],
                             preferred_element_type=jnp.float32) + bo_ref[...]


def _head(acc, z2, dinv, b2, ct, wo, bo):
    return pl.pallas_call(
        _head_body,
        grid=(_GRID,),
        in_specs=[
            pl.BlockSpec((1, _R, D), lambda i: (0, i, 0)),
            pl.BlockSpec((1, _R, D), lambda i: (1, i, 0)),
            pl.BlockSpec((_R, D), lambda i: (i, 0)),
            pl.BlockSpec((_R, 1), lambda i: (i, 0)),
            pl.BlockSpec((1, D), lambda i: (0, 0)),
            pl.BlockSpec((D, D), lambda i: (0, 0)),
            pl.BlockSpec((D, D), lambda i: (0, 0)),
            pl.BlockSpec((1, D), lambda i: (0, 0)),
        ],
        out_specs=pl.BlockSpec((1, D), lambda i: (0, 0)),
        out_shape=jax.ShapeDtypeStruct((1, D), jnp.float32),
        scratch_shapes=[pltpu.VMEM((1, D), jnp.float32)],
    )(acc, acc, z2, dinv, b2, ct, wo, bo)


def kernel(x, edge_index, W_embed, W1, b1, W2, b2, centroids, W_out, b_out):
    ei5 = edge_index.astype(jnp.int32).reshape(2, 32, NB_TILE // CHUNK,
                                               CHUNK, BATCH)
    zf = jnp.zeros((N_PAD, D), jnp.float32)
    zc = jnp.zeros((CNT_PAD,), jnp.float32)

    y = _pre(x, W_embed, W1)
    cnt = _deg(ei5, zc)
    cnt3 = cnt[:, :N].reshape(2, N, 1)
    z1, dinv = _scale(y, cnt3)

    acc1 = _scat(z1, ei5, zf)
    z2 = _mid(acc1, z1, dinv, b1.reshape(1, D), W2)
    acc2 = _scat(z2, ei5, zf)

    cpad = jnp.zeros((D, D), jnp.float32).at[:centroids.shape[0]].set(centroids)
    wo = jnp.zeros((D, D), jnp.float32).at[:W_out.shape[0], :W_out.shape[1]].set(W_out)
    bo = jnp.zeros((1, D), jnp.float32).at[0, :b_out.shape[0]].set(b_out)
    o = _head(acc2, z2, dinv, b2.reshape(1, D), cpad.T, wo, bo)
    return o[0, :b_out.shape[0]]


# speedup vs baseline: 30.3425x; 1.0347x over previous
import jax
import jax.numpy as jnp
from jax import lax
from jax.experimental import pallas as pl
from jax.experimental.pallas import tpu as pltpu
from jax.experimental.pallas import tpu_sc as plsc

N = 10000
D = 128
E = 320000
EPT = E // 32
BATCH = 80
NB_TILE = EPT // BATCH
CHUNK = 25
NCH = NB_TILE // CHUNK
N_PAD = 10112
ROWS_TILE = N_PAD // 16
CNT_PAD = 10240
CNT_TILE = CNT_PAD // 16

_MESH = plsc.VectorSubcoreMesh(core_axis_name="c", subcore_axis_name="s")


def _deg_body(ei_hbm, zero_hbm, cnt_out, dstv, onesv, cnt_sh):
    c = lax.axis_index("c")
    s = lax.axis_index("s")
    wid = s * 2 + c
    pltpu.sync_copy(zero_hbm.at[pl.ds(s * CNT_TILE, CNT_TILE)],
                    cnt_sh.at[pl.ds(s * CNT_TILE, CNT_TILE)])
    for i in range(BATCH // 16):
        onesv[pl.ds(i * 16, 16)] = jnp.ones((16,), jnp.float32)
    pltpu.sync_copy(ei_hbm.at[1].at[wid], dstv)
    plsc.subcore_barrier()

    def body(j, carry):
        pltpu.sync_copy(onesv, cnt_sh.at[dstv.at[j // CHUNK].at[j % CHUNK]],
                        add=True)
        return carry

    lax.fori_loop(0, NB_TILE, body, 0)
    plsc.subcore_barrier()
    pltpu.sync_copy(cnt_sh.at[pl.ds(s * CNT_TILE, CNT_TILE)],
                    cnt_out.at[c].at[pl.ds(s * CNT_TILE, CNT_TILE)])


_deg = pl.kernel(
    _deg_body,
    out_type=jax.ShapeDtypeStruct((2, CNT_PAD), jnp.float32),
    mesh=_MESH,
    scratch_types=[
        pltpu.VMEM((NCH, CHUNK, BATCH), jnp.int32),
        pltpu.VMEM((BATCH,), jnp.float32),
        pltpu.VMEM_SHARED((CNT_PAD,), jnp.float32),
    ],
)


def _scat_body(z_hbm, ei_hbm, zero_hbm, out_hbm,
               srcv, dstv, rows, acc_sh, sem0, sem1):
    c = lax.axis_index("c")
    s = lax.axis_index("s")
    wid = s * 2 + c
    pltpu.sync_copy(ei_hbm.at[0].at[wid].at[0], srcv.at[0])
    pltpu.sync_copy(ei_hbm.at[1].at[wid].at[0], dstv)
    pltpu.async_copy(z_hbm.at[srcv.at[0].at[0]], rows.at[0], sem0)
    pltpu.async_copy(z_hbm.at[srcv.at[0].at[1]], rows.at[1], sem1)
    pltpu.sync_copy(zero_hbm, acc_sh.at[pl.ds(s * ROWS_TILE, ROWS_TILE)])
    plsc.subcore_barrier()

    order = [(0, sem0), (1, sem1)]
    for c5 in range(NCH):
        si = c5 % 2
        (b0, s0), (b1, s1) = order

        def pair(p, carry, b0=b0, s0=s0, b1=b1, s1=s1, si=si):
            for bb, (bf, sm) in enumerate(((b0, s0), (b1, s1))):
                j = 2 * p + bb
                pltpu.make_async_copy(z_hbm.at[srcv.at[si].at[j]],
                                      rows.at[bf], sm).wait()
                pltpu.sync_copy(rows.at[bf], acc_sh.at[dstv.at[j]], add=True)
                nxt = j + 2

                @pl.when(nxt < CHUNK)
                def _():
                    pltpu.async_copy(z_hbm.at[srcv.at[si].at[nxt]],
                                     rows.at[bf], sm)
            return carry

        lax.fori_loop(0, CHUNK // 2, pair, 0)
        if c5 + 1 < NCH:
            pltpu.sync_copy(ei_hbm.at[0].at[wid].at[c5 + 1],
                            srcv.at[1 - si])
        pltpu.make_async_copy(z_hbm.at[srcv.at[si].at[CHUNK - 1]],
                              rows.at[b0], s0).wait()
        if c5 + 1 < NCH:
            pltpu.async_copy(z_hbm.at[srcv.at[1 - si].at[0]],
                             rows.at[b1], s1)
        pltpu.sync_copy(rows.at[b0], acc_sh.at[dstv.at[CHUNK - 1]], add=True)
        if c5 + 1 < NCH:
            pltpu.sync_copy(ei_hbm.at[1].at[wid].at[c5 + 1], dstv)
            pltpu.async_copy(z_hbm.at[srcv.at[1 - si].at[1]],
                             rows.at[b0], s0)
            order = [(b1, s1), (b0, s0)]

    plsc.subcore_barrier()
    pltpu.sync_copy(acc_sh.at[pl.ds(s * ROWS_TILE, ROWS_TILE)],
                    out_hbm.at[c].at[pl.ds(s * ROWS_TILE, ROWS_TILE)])


_scat = pl.kernel(
    _scat_body,
    out_type=jax.ShapeDtypeStruct((2, N_PAD, D), jnp.float32),
    mesh=_MESH,
    scratch_types=[
        pltpu.VMEM((2, CHUNK, BATCH), jnp.int32),
        pltpu.VMEM((CHUNK, BATCH), jnp.int32),
        pltpu.VMEM((2, BATCH, D), jnp.float32),
        pltpu.VMEM_SHARED((N_PAD, D), jnp.float32),
        pltpu.SemaphoreType.DMA,
        pltpu.SemaphoreType.DMA,
    ],
)

_R = 1000
_GRID = N // _R


def _pre_body(x_ref, we_ref, w1_ref, y_ref):
    h0 = jnp.dot(x_ref[...], we_ref[...], preferred_element_type=jnp.float32)
    y_ref[...] = jnp.dot(h0, w1_ref[...], preferred_element_type=jnp.float32)


def _pre(x, we, w1):
    return pl.pallas_call(
        _pre_body,
        grid=(_GRID,),
        in_specs=[
            pl.BlockSpec((_R, D), lambda i: (i, 0)),
            pl.BlockSpec((D, D), lambda i: (0, 0)),
            pl.BlockSpec((D, D), lambda i: (0, 0)),
        ],
        out_specs=pl.BlockSpec((_R, D), lambda i: (i, 0)),
        out_shape=jax.ShapeDtypeStruct((N, D), jnp.float32),
    )(x, we, w1)


def _scale_body(y_ref, cnt_ref, z1_ref, dinv_ref):
    dinv = lax.rsqrt(1.0 + cnt_ref[0] + cnt_ref[1])
    z1_ref[...] = y_ref[...] * dinv
    dinv_ref[...] = dinv


def _scale(y, cnt3):
    return pl.pallas_call(
        _scale_body,
        grid=(_GRID,),
        in_specs=[
            pl.BlockSpec((_R, D), lambda i: (i, 0)),
            pl.BlockSpec((2, _R, 1), lambda i: (0, i, 0)),
        ],
        out_specs=[
            pl.BlockSpec((_R, D), lambda i: (i, 0)),
            pl.BlockSpec((_R, 1), lambda i: (i, 0)),
        ],
        out_shape=[
            jax.ShapeDtypeStruct((N, D), jnp.float32),
            jax.ShapeDtypeStruct((N, 1), jnp.float32),
        ],
    )(y, cnt3)


def _mid_body(aa_ref, ab_ref, z1_ref, dinv_ref, b1_ref, w2_ref, z2_ref):
    dinv = dinv_ref[...]
    h1 = jnp.maximum(
        (aa_ref[0] + ab_ref[0] + z1_ref[...]) * dinv + b1_ref[...], 0.0)
    z2_ref[...] = jnp.dot(h1, w2_ref[...],
                          preferred_element_type=jnp.float32) * dinv


def _mid(acc, z1, dinv, b1, w2):
    return pl.pallas_call(
        _mid_body,
        grid=(_GRID,),
        in_specs=[
            pl.BlockSpec((1, _R, D), lambda i: (0, i, 0)),
            pl.BlockSpec((1, _R, D), lambda i: (1, i, 0)),
            pl.BlockSpec((_R, D), lambda i: (i, 0)),
            pl.BlockSpec((_R, 1), lambda i: (i, 0)),
            pl.BlockSpec((1, D), lambda i: (0, 0)),
            pl.BlockSpec((D, D), lambda i: (0, 0)),
        ],
        out_specs=pl.BlockSpec((_R, D), lambda i: (i, 0)),
        out_shape=jax.ShapeDtypeStruct((N, D), jnp.float32),
    )(acc, acc, z1, dinv, b1, w2)


def _head_body(aa_ref, ab_ref, z2_ref, dinv_ref, b2_ref, ct_ref, wo_ref,
               bo_ref, o_ref, acc_ref):
    i = pl.program_id(0)
    h2 = jnp.maximum(
        (aa_ref[0] + ab_ref[0] + z2_ref[...]) * dinv_ref[...]
        + b2_ref[...], 0.0)
    g = jnp.dot(h2, ct_ref[...], preferred_element_type=jnp.float32)
    rn = jnp.sum(h2 * h2, axis=1, keepdims=True)
    cn = jnp.sum(ct_ref[...] * ct_ref[...], axis=0, keepdims=True)
    dist = jnp.sqrt(jnp.clip(rn + cn - 2.0 * g, 1e-12, None))
    colsum = jnp.sum(dist, axis=0, keepdims=True)

    @pl.when(i == 0)
    def _():
        acc_ref[...] = jnp.zeros_like(acc_ref)

    acc_ref[...] += colsum

    @pl.when(i == pl.num_programs(0) - 1)
    def _():
        o_ref[...] = jnp.dot(acc_ref[...] * (1.0 / N), wo_ref[...],
                             preferred_element_type=jnp.float32) + bo_ref[...]


def _head(acc, z2, dinv, b2, ct, wo, bo):
    return pl.pallas_call(
        _head_body,
        grid=(_GRID,),
        in_specs=[
            pl.BlockSpec((1, _R, D), lambda i: (0, i, 0)),
            pl.BlockSpec((1, _R, D), lambda i: (1, i, 0)),
            pl.BlockSpec((_R, D), lambda i: (i, 0)),
            pl.BlockSpec((_R, 1), lambda i: (i, 0)),
            pl.BlockSpec((1, D), lambda i: (0, 0)),
            pl.BlockSpec((D, D), lambda i: (0, 0)),
            pl.BlockSpec((D, D), lambda i: (0, 0)),
            pl.BlockSpec((1, D), lambda i: (0, 0)),
        ],
        out_specs=pl.BlockSpec((1, D), lambda i: (0, 0)),
        out_shape=jax.ShapeDtypeStruct((1, D), jnp.float32),
        scratch_shapes=[pltpu.VMEM((1, D), jnp.float32)],
    )(acc, acc, z2, dinv, b2, ct, wo, bo)


def kernel(x, edge_index, W_embed, W1, b1, W2, b2, centroids, W_out, b_out):
    ei32 = edge_index.astype(jnp.int32)
    ei5 = ei32.reshape(2, 32, NCH, CHUNK, BATCH)
    zrow = jnp.zeros((ROWS_TILE, D), jnp.float32)
    zc = jnp.zeros((CNT_PAD,), jnp.float32)

    y = _pre(x, W_embed, W1)
    cnt = _deg(ei5, zc)
    cnt3 = cnt[:, :N].reshape(2, N, 1)
    z1, dinv = _scale(y, cnt3)

    acc1 = _scat(z1, ei5, zrow)
    z2 = _mid(acc1, z1, dinv, b1.reshape(1, D), W2)
    acc2 = _scat(z2, ei5, zrow)

    cpad = jnp.zeros((D, D), jnp.float32).at[:centroids.shape[0]].set(centroids)
    wo = jnp.zeros((D, D), jnp.float32).at[:W_out.shape[0], :W_out.shape[1]].set(W_out)
    bo = jnp.zeros((1, D), jnp.float32).at[0, :b_out.shape[0]].set(b_out)
    o = _head(acc2, z2, dinv, b2.reshape(1, D), cpad.T, wo, bo)
    return o[0, :b_out.shape[0]]


# speedup vs baseline: 31.5041x; 1.0383x over previous
import jax
import jax.numpy as jnp
from jax import lax
from jax.experimental import pallas as pl
from jax.experimental.pallas import tpu as pltpu
from jax.experimental.pallas import tpu_sc as plsc

N = 10000
D = 128
E = 320000
EPT = E // 32
BATCH = 80
NB_TILE = EPT // BATCH
CHUNK = 25
NCH = NB_TILE // CHUNK
N_PAD = 10112
ROWS_TILE = N_PAD // 16
CNT_PAD = 10240
CNT_TILE = CNT_PAD // 16

_MESH = plsc.VectorSubcoreMesh(core_axis_name="c", subcore_axis_name="s")


def _deg_body(ei_hbm, zero_hbm, cnt_out, dstv, onesv, cnt_sh):
    c = lax.axis_index("c")
    s = lax.axis_index("s")
    wid = s * 2 + c
    pltpu.sync_copy(ei_hbm.at[1].at[pl.ds(wid * 9984, 9984)],
                    dstv.at[pl.ds(0, 9984)])

    @pl.when(wid < 4)
    def _():
        pltpu.sync_copy(ei_hbm.at[1].at[pl.ds(32 * 9984 + wid * 128, 128)],
                        dstv.at[pl.ds(9984, 128)])

    pltpu.sync_copy(zero_hbm.at[pl.ds(s * CNT_TILE, CNT_TILE)],
                    cnt_sh.at[pl.ds(s * CNT_TILE, CNT_TILE)])
    for i in range(128 // 16):
        onesv[pl.ds(i * 16, 16)] = jnp.ones((16,), jnp.float32)
    plsc.subcore_barrier()

    def body(j, carry):
        pltpu.sync_copy(onesv.at[pl.ds(0, 96)],
                        cnt_sh.at[dstv.at[pl.ds(j * 96, 96)]], add=True)
        return carry

    lax.fori_loop(0, 9984 // 96, body, 0)

    @pl.when(wid < 4)
    def _():
        pltpu.sync_copy(onesv, cnt_sh.at[dstv.at[pl.ds(9984, 128)]], add=True)

    plsc.subcore_barrier()
    pltpu.sync_copy(cnt_sh.at[pl.ds(s * CNT_TILE, CNT_TILE)],
                    cnt_out.at[c].at[pl.ds(s * CNT_TILE, CNT_TILE)])


_deg = pl.kernel(
    _deg_body,
    out_type=jax.ShapeDtypeStruct((2, CNT_PAD), jnp.float32),
    mesh=_MESH,
    scratch_types=[
        pltpu.VMEM((9984 + 128,), jnp.int32),
        pltpu.VMEM((128,), jnp.float32),
        pltpu.VMEM_SHARED((CNT_PAD,), jnp.float32),
    ],
)


def _scat_body(z_hbm, ei_hbm, zero_hbm, out_hbm,
               srcv, dstv, rows, acc_sh, sem0, sem1):
    c = lax.axis_index("c")
    s = lax.axis_index("s")
    wid = s * 2 + c
    pltpu.sync_copy(ei_hbm.at[0].at[wid].at[0], srcv.at[0])
    pltpu.sync_copy(ei_hbm.at[1].at[wid].at[0], dstv)
    pltpu.async_copy(z_hbm.at[srcv.at[0].at[0]], rows.at[0], sem0)
    pltpu.async_copy(z_hbm.at[srcv.at[0].at[1]], rows.at[1], sem1)
    pltpu.sync_copy(zero_hbm, acc_sh.at[pl.ds(s * ROWS_TILE, ROWS_TILE)])
    plsc.subcore_barrier()

    order = [(0, sem0), (1, sem1)]
    for c5 in range(NCH):
        si = c5 % 2
        (b0, s0), (b1, s1) = order

        def pair(p, carry, b0=b0, s0=s0, b1=b1, s1=s1, si=si):
            for bb, (bf, sm) in enumerate(((b0, s0), (b1, s1))):
                j = 2 * p + bb
                pltpu.make_async_copy(z_hbm.at[srcv.at[si].at[j]],
                                      rows.at[bf], sm).wait()
                pltpu.sync_copy(rows.at[bf], acc_sh.at[dstv.at[j]], add=True)
                nxt = j + 2

                @pl.when(nxt < CHUNK)
                def _():
                    pltpu.async_copy(z_hbm.at[srcv.at[si].at[nxt]],
                                     rows.at[bf], sm)
            return carry

        lax.fori_loop(0, CHUNK // 2, pair, 0)
        if c5 + 1 < NCH:
            pltpu.sync_copy(ei_hbm.at[0].at[wid].at[c5 + 1],
                            srcv.at[1 - si])
        pltpu.make_async_copy(z_hbm.at[srcv.at[si].at[CHUNK - 1]],
                              rows.at[b0], s0).wait()
        if c5 + 1 < NCH:
            pltpu.async_copy(z_hbm.at[srcv.at[1 - si].at[0]],
                             rows.at[b1], s1)
        pltpu.sync_copy(rows.at[b0], acc_sh.at[dstv.at[CHUNK - 1]], add=True)
        if c5 + 1 < NCH:
            pltpu.sync_copy(ei_hbm.at[1].at[wid].at[c5 + 1], dstv)
            pltpu.async_copy(z_hbm.at[srcv.at[1 - si].at[1]],
                             rows.at[b0], s0)
            order = [(b1, s1), (b0, s0)]

    plsc.subcore_barrier()
    pltpu.sync_copy(acc_sh.at[pl.ds(s * ROWS_TILE, ROWS_TILE)],
                    out_hbm.at[c].at[pl.ds(s * ROWS_TILE, ROWS_TILE)])


_scat = pl.kernel(
    _scat_body,
    out_type=jax.ShapeDtypeStruct((2, N_PAD, D), jnp.float32),
    mesh=_MESH,
    scratch_types=[
        pltpu.VMEM((2, CHUNK, BATCH), jnp.int32),
        pltpu.VMEM((CHUNK, BATCH), jnp.int32),
        pltpu.VMEM((2, BATCH, D), jnp.float32),
        pltpu.VMEM_SHARED((N_PAD, D), jnp.float32),
        pltpu.SemaphoreType.DMA,
        pltpu.SemaphoreType.DMA,
    ],
)

_R = 2000
_GRID = N // _R


def _pre_body(x_ref, we_ref, w1_ref, y_ref):
    h0 = jnp.dot(x_ref[...], we_ref[...], preferred_element_type=jnp.float32)
    y_ref[...] = jnp.dot(h0, w1_ref[...], preferred_element_type=jnp.float32)


def _pre(x, we, w1):
    return pl.pallas_call(
        _pre_body,
        grid=(_GRID,),
        in_specs=[
            pl.BlockSpec((_R, D), lambda i: (i, 0)),
            pl.BlockSpec((D, D), lambda i: (0, 0)),
            pl.BlockSpec((D, D), lambda i: (0, 0)),
        ],
        out_specs=pl.BlockSpec((_R, D), lambda i: (i, 0)),
        out_shape=jax.ShapeDtypeStruct((N, D), jnp.float32),
    )(x, we, w1)


def _scale_body(y_ref, cnt_ref, z1_ref, dinv_ref):
    dinv = lax.rsqrt(1.0 + cnt_ref[0] + cnt_ref[1])
    z1_ref[...] = y_ref[...] * dinv
    dinv_ref[...] = dinv


def _scale(y, cnt3):
    return pl.pallas_call(
        _scale_body,
        grid=(_GRID,),
        in_specs=[
            pl.BlockSpec((_R, D), lambda i: (i, 0)),
            pl.BlockSpec((2, _R, 1), lambda i: (0, i, 0)),
        ],
        out_specs=[
            pl.BlockSpec((_R, D), lambda i: (i, 0)),
            pl.BlockSpec((_R, 1), lambda i: (i, 0)),
        ],
        out_shape=[
            jax.ShapeDtypeStruct((N, D), jnp.float32),
            jax.ShapeDtypeStruct((N, 1), jnp.float32),
        ],
    )(y, cnt3)


def _mid_body(aa_ref, ab_ref, z1_ref, dinv_ref, b1_ref, w2_ref, z2_ref):
    dinv = dinv_ref[...]
    h1 = jnp.maximum(
        (aa_ref[0] + ab_ref[0] + z1_ref[...]) * dinv + b1_ref[...], 0.0)
    z2_ref[...] = jnp.dot(h1, w2_ref[...],
                          preferred_element_type=jnp.float32) * dinv


def _mid(acc, z1, dinv, b1, w2):
    return pl.pallas_call(
        _mid_body,
        grid=(_GRID,),
        in_specs=[
            pl.BlockSpec((1, _R, D), lambda i: (0, i, 0)),
            pl.BlockSpec((1, _R, D), lambda i: (1, i, 0)),
            pl.BlockSpec((_R, D), lambda i: (i, 0)),
            pl.BlockSpec((_R, 1), lambda i: (i, 0)),
            pl.BlockSpec((1, D), lambda i: (0, 0)),
            pl.BlockSpec((D, D), lambda i: (0, 0)),
        ],
        out_specs=pl.BlockSpec((_R, D), lambda i: (i, 0)),
        out_shape=jax.ShapeDtypeStruct((N, D), jnp.float32),
    )(acc, acc, z1, dinv, b1, w2)


def _head_body(aa_ref, ab_ref, z2_ref, dinv_ref, b2_ref, ct_ref, wo_ref,
               bo_ref, o_ref, acc_ref):
    i = pl.program_id(0)
    h2 = jnp.maximum(
        (aa_ref[0] + ab_ref[0] + z2_ref[...]) * dinv_ref[...]
        + b2_ref[...], 0.0)
    g = jnp.dot(h2, ct_ref[...], preferred_element_type=jnp.float32)
    rn = jnp.sum(h2 * h2, axis=1, keepdims=True)
    cn = jnp.sum(ct_ref[...] * ct_ref[...], axis=0, keepdims=True)
    dist = jnp.sqrt(jnp.clip(rn + cn - 2.0 * g, 1e-12, None))
    colsum = jnp.sum(dist, axis=0, keepdims=True)

    @pl.when(i == 0)
    def _():
        acc_ref[...] = jnp.zeros_like(acc_ref)

    acc_ref[...] += colsum

    @pl.when(i == pl.num_programs(0) - 1)
    def _():
        o_ref[...] = jnp.dot(acc_ref[...] * (1.0 / N), wo_ref[...],
                             preferred_element_type=jnp.float32) + bo_ref[...]


def _head(acc, z2, dinv, b2, ct, wo, bo):
    return pl.pallas_call(
        _head_body,
        grid=(_GRID,),
        in_specs=[
            pl.BlockSpec((1, _R, D), lambda i: (0, i, 0)),
            pl.BlockSpec((1, _R, D), lambda i: (1, i, 0)),
            pl.BlockSpec((_R, D), lambda i: (i, 0)),
            pl.BlockSpec((_R, 1), lambda i: (i, 0)),
            pl.BlockSpec((1, D), lambda i: (0, 0)),
            pl.BlockSpec((D, D), lambda i: (0, 0)),
            pl.BlockSpec((D, D), lambda i: (0, 0)),
            pl.BlockSpec((1, D), lambda i: (0, 0)),
        ],
        out_specs=pl.BlockSpec((1, D), lambda i: (0, 0)),
        out_shape=jax.ShapeDtypeStruct((1, D), jnp.float32),
        scratch_shapes=[pltpu.VMEM((1, D), jnp.float32)],
    )(acc, acc, z2, dinv, b2, ct, wo, bo)


def kernel(x, edge_index, W_embed, W1, b1, W2, b2, centroids, W_out, b_out):
    ei32 = edge_index.astype(jnp.int32)
    ei5 = ei32.reshape(2, 32, NCH, CHUNK, BATCH)
    zrow = jnp.zeros((ROWS_TILE, D), jnp.float32)
    zc = jnp.zeros((CNT_PAD,), jnp.float32)

    y = _pre(x, W_embed, W1)
    cnt = _deg(ei32, zc)
    cnt3 = cnt[:, :N].reshape(2, N, 1)
    z1, dinv = _scale(y, cnt3)

    acc1 = _scat(z1, ei5, zrow)
    z2 = _mid(acc1, z1, dinv, b1.reshape(1, D), W2)
    acc2 = _scat(z2, ei5, zrow)

    cpad = jnp.zeros((D, D), jnp.float32).at[:centroids.shape[0]].set(centroids)
    wo = jnp.zeros((D, D), jnp.float32).at[:W_out.shape[0], :W_out.shape[1]].set(W_out)
    bo = jnp.zeros((1, D), jnp.float32).at[0, :b_out.shape[0]].set(b_out)
    o = _head(acc2, z2, dinv, b2.reshape(1, D), cpad.T, wo, bo)
    return o[0, :b_out.shape[0]]
